# baseline (speedup 1.0000x reference)
import functools

import jax
import jax.numpy as jnp
from jax import lax
from jax.experimental import pallas as pl
from jax.experimental.pallas import tpu as pltpu
from jax.experimental.pallas import tpu_sc as plsc

N = 10000
E = 320000
D_IN = 128
D_HID = 64
N_CLASSES = 47
K_HOPS = 8
C1 = 0.5
C2 = 0.5

NPAD = 10240
C = 128
CPW = 80
EPAD = 32 * CPW * C
RZT = NPAD // 16
RPT = NPAD // 32
NSC = 16


def _mesh():
    return plsc.VectorSubcoreMesh(
        core_axis_name="c", subcore_axis_name="s", num_cores=2, num_subcores=NSC
    )


NB = 4
LD = 2


def _make_prop(d):
    @functools.partial(
        pl.kernel,
        out_type=jax.ShapeDtypeStruct((2, NPAD, d), jnp.float32),
        mesh=_mesh(),
        scratch_types=[
            pltpu.VMEM((CPW, C), jnp.int32),
            pltpu.VMEM((CPW, C), jnp.int32),
            pltpu.VMEM((NB, C, d), jnp.float32),
            pltpu.VMEM_SHARED((NPAD, d), jnp.float32),
            pltpu.SemaphoreType.DMA((NB,)),
            pltpu.SemaphoreType.DMA((NB,)),
        ],
        compiler_params=pltpu.CompilerParams(use_tc_tiling_on_sc=False),
        name=f"vgcn_prop_{d}",
    )
    def prop(zs_hbm, src_hbm, dst_hbm, zer_hbm, acc_out,
             sidx, didx, gbufs, acc_sh, gsem, ssem):
        c = lax.axis_index("c")
        s = lax.axis_index("s")
        pltpu.sync_copy(zer_hbm, acc_sh.at[pl.ds(s * RZT, RZT)])
        plsc.subcore_barrier()
        w = c * NSC + s
        pltpu.sync_copy(src_hbm.at[w], sidx)
        pltpu.sync_copy(dst_hbm.at[w], didx)

        def g_start(j, b):
            pltpu.async_copy(zs_hbm.at[sidx.at[j]], gbufs.at[b], gsem.at[b])

        def g_wait(j, b):
            pltpu.make_async_copy(zs_hbm.at[sidx.at[j]], gbufs.at[b],
                                  gsem.at[b]).wait()

        def s_start(j, b):
            pltpu.async_copy(gbufs.at[b], acc_sh.at[didx.at[j]], ssem.at[b],
                             add=True)

        def s_wait(j, b):
            pltpu.make_async_copy(gbufs.at[b], acc_sh.at[didx.at[j]],
                                  ssem.at[b]).wait()

        def steady(j, with_gather=True):
            b = j % NB if isinstance(j, int) else None
            assert b is not None
            b2 = (j - LD) % NB
            s_wait(j - NB, b)
            if with_gather:
                g_start(j, b)
            g_wait(j - LD, b2)
            s_start(j - LD, b2)

        def steady_traced(j, boff):
            b2 = (boff - LD) % NB
            s_wait(j - NB, boff)
            g_start(j, boff)
            g_wait(j - LD, b2)
            s_start(j - LD, b2)

        g_start(0, 0)
        g_start(1, 1)
        g_start(2, 2)
        g_wait(0, 0)
        s_start(0, 0)
        g_start(3, 3)
        g_wait(1, 1)
        s_start(1, 1)
        steady(4)
        steady(5)

        ngrp = (CPW - 2 - 6) // NB

        def group(g, carry):
            j0 = 6 + g * NB
            for bb in range(NB):
                steady_traced(j0 + bb, (6 + bb) % NB)
            return carry

        lax.fori_loop(0, ngrp, group, 0)

        steady(CPW - 2)
        steady(CPW - 1)
        steady(CPW, with_gather=False)
        steady(CPW + 1, with_gather=False)
        s_wait(CPW - 2, (CPW - 2) % NB)
        s_wait(CPW - 1, (CPW - 1) % NB)

        plsc.subcore_barrier()
        pltpu.sync_copy(acc_sh.at[pl.ds(s * RZT, RZT)],
                        acc_out.at[c, pl.ds(s * RZT, RZT)])

    return prop


_prop64 = _make_prop(D_HID)
_prop48 = _make_prop(48)
_prop16 = _make_prop(16)


def _mlp(x, w, b, deg_acc, dout):
    bm = 1024
    kd = x.shape[1]

    def body(x_ref, w_ref, b_ref, dacc_ref, out_ref, outs_ref, dis_ref,
             d2_ref):
        deg = dacc_ref[0, :, :1] + dacc_ref[1, :, :1]
        dis = jnp.where(deg > 0.5, lax.rsqrt(deg), 0.0)
        acc = jnp.dot(x_ref[...], w_ref[...],
                      preferred_element_type=jnp.float32) + b_ref[...]
        out_ref[...] = acc
        outs_ref[...] = acc * dis
        dis_ref[...] = dis
        d2_ref[...] = dis * dis

    return pl.pallas_call(
        body,
        grid=(NPAD // bm,),
        in_specs=[
            pl.BlockSpec((bm, kd), lambda i: (i, 0)),
            pl.BlockSpec((kd, dout), lambda i: (0, 0)),
            pl.BlockSpec((1, dout), lambda i: (0, 0)),
            pl.BlockSpec((2, bm, 16), lambda i: (0, i, 0)),
        ],
        out_specs=[pl.BlockSpec((bm, dout), lambda i: (i, 0))] * 2
        + [pl.BlockSpec((bm, 1), lambda i: (i, 0))] * 2,
        out_shape=[jax.ShapeDtypeStruct((NPAD, dout), jnp.float32)] * 2
        + [jax.ShapeDtypeStruct((NPAD, 1), jnp.float32)] * 2,
    )(x, w, b.reshape(1, dout), deg_acc)


def _update(acc, a, b2d, dout):
    bm = 1024

    def body(acc_ref, a_ref, b_ref, o_ref):
        o_ref[...] = C1 * a_ref[...] + C2 * b_ref[...] * (acc_ref[0] + acc_ref[1])

    return pl.pallas_call(
        body,
        grid=(NPAD // bm,),
        in_specs=[
            pl.BlockSpec((2, bm, dout), lambda i: (0, i, 0)),
            pl.BlockSpec((bm, dout), lambda i: (i, 0)),
            pl.BlockSpec((bm, 1), lambda i: (i, 0)),
        ],
        out_specs=pl.BlockSpec((bm, dout), lambda i: (i, 0)),
        out_shape=jax.ShapeDtypeStruct((NPAD, dout), jnp.float32),
    )(acc, a, b2d)


def _block(prop, zs0, srcp, dstp, zer, ini, ini_s, dis2d, d22d, d):
    zs = zs0
    for _ in range(K_HOPS - 1):
        acc = prop(zs, srcp, dstp, zer)
        zs = _update(acc, ini_s, d22d, d)
    acc = prop(zs, srcp, dstp, zer)
    return _update(acc, ini, dis2d, d)


def kernel(features, edge_index, W1, b1, W2, b2):
    src = edge_index[0].astype(jnp.int32)
    dst = edge_index[1].astype(jnp.int32)
    srcp = jnp.concatenate(
        [src, jnp.full((EPAD - E,), N, jnp.int32)]).reshape(32, CPW, C)
    dstp = jnp.concatenate(
        [dst, jnp.full((EPAD - E,), NPAD - 1, jnp.int32)]).reshape(32, CPW, C)

    ones16 = jnp.broadcast_to(
        (jnp.arange(NPAD) < N).astype(jnp.float32)[:, None], (NPAD, 16))
    zer16 = jnp.zeros((RZT, 16), jnp.float32)
    zer64 = jnp.zeros((RZT, D_HID), jnp.float32)
    zer48 = jnp.zeros((RZT, 48), jnp.float32)

    deg_acc = _prop16(ones16, srcp, dstp, zer16)

    featp = jnp.pad(features, ((0, NPAD - N), (0, 0)))
    ini1, ini1_s, dis2d, d22d = _mlp(featp, W1, b1, deg_acc, D_HID)
    h = _block(_prop64, ini1_s, srcp, dstp, zer64, ini1, ini1_s, dis2d, d22d,
               D_HID)

    w2p = jnp.pad(W2, ((0, 0), (0, 1)))
    b2p = jnp.pad(b2, (0, 1))
    ini2, ini2_s, _, _ = _mlp(h, w2p, b2p, deg_acc, 48)
    out = _block(_prop48, ini2_s, srcp, dstp, zer48, ini2, ini2_s, dis2d, d22d,
                 48)
    return out[:N, :N_CLASSES]


# speedup vs baseline: 1.0042x; 1.0042x over previous
import functools

import jax
import jax.numpy as jnp
from jax import lax
from jax.experimental import pallas as pl
from jax.experimental.pallas import tpu as pltpu
from jax.experimental.pallas import tpu_sc as plsc

N = 10000
E = 320000
D_IN = 128
D_HID = 64
N_CLASSES = 47
K_HOPS = 8
C1 = 0.5
C2 = 0.5

NPAD = 10240
C = 128
CPW = 80
EPAD = 32 * CPW * C
RZT = NPAD // 16
RPT = NPAD // 32
NSC = 16


def _mesh():
    return plsc.VectorSubcoreMesh(
        core_axis_name="c", subcore_axis_name="s", num_cores=2, num_subcores=NSC
    )


NB = 8
LD = 5


def _make_prop(d):
    @functools.partial(
        pl.kernel,
        out_type=jax.ShapeDtypeStruct((2, NPAD, d), jnp.float32),
        mesh=_mesh(),
        scratch_types=[
            pltpu.VMEM((CPW, C), jnp.int32),
            pltpu.VMEM((CPW, C), jnp.int32),
            pltpu.VMEM((NB, C, d), jnp.float32),
            pltpu.VMEM_SHARED((NPAD, d), jnp.float32),
            pltpu.SemaphoreType.DMA((NB,)),
            pltpu.SemaphoreType.DMA((NB,)),
        ],
        compiler_params=pltpu.CompilerParams(use_tc_tiling_on_sc=False),
        name=f"vgcn_prop_{d}",
    )
    def prop(zs_hbm, src_hbm, dst_hbm, zer_hbm, acc_out,
             sidx, didx, gbufs, acc_sh, gsem, ssem):
        c = lax.axis_index("c")
        s = lax.axis_index("s")
        pltpu.sync_copy(zer_hbm, acc_sh.at[pl.ds(s * RZT, RZT)])
        plsc.subcore_barrier()
        w = c * NSC + s
        pltpu.sync_copy(src_hbm.at[w], sidx)
        pltpu.sync_copy(dst_hbm.at[w], didx)

        def g_start(j, b):
            pltpu.async_copy(zs_hbm.at[sidx.at[j]], gbufs.at[b], gsem.at[b])

        def g_wait(j, b):
            pltpu.make_async_copy(zs_hbm.at[sidx.at[j]], gbufs.at[b],
                                  gsem.at[b]).wait()

        def s_start(j, b):
            pltpu.async_copy(gbufs.at[b], acc_sh.at[didx.at[j]], ssem.at[b],
                             add=True)

        def s_wait(j, b):
            pltpu.make_async_copy(gbufs.at[b], acc_sh.at[didx.at[j]],
                                  ssem.at[b]).wait()

        def pos(j, b, drain, gather, consume):
            if drain:
                s_wait(j - NB, b)
            if gather:
                g_start(j, b)
            if consume:
                b2 = (b - LD) % NB
                g_wait(j - LD, b2)
                s_start(j - LD, b2)

        for j in range(NB):
            pos(j, j % NB, False, True, j >= LD)

        def group(g, carry):
            j0 = NB + g * NB
            for bb in range(NB):
                pos(j0 + bb, bb, True, True, True)
            return carry

        lax.fori_loop(0, (CPW - NB) // NB, group, 0)

        for j in range(CPW, CPW + LD):
            pos(j, j % NB, False, False, True)
        for ch in range(CPW - NB, CPW):
            s_wait(ch, ch % NB)

        plsc.subcore_barrier()
        pltpu.sync_copy(acc_sh.at[pl.ds(s * RZT, RZT)],
                        acc_out.at[c, pl.ds(s * RZT, RZT)])

    return prop


_prop64 = _make_prop(D_HID)
_prop48 = _make_prop(48)
_prop16 = _make_prop(16)


def _mlp(x, w, b, deg_acc, dout):
    bm = 1024
    kd = x.shape[1]

    def body(x_ref, w_ref, b_ref, dacc_ref, out_ref, outs_ref, dis_ref,
             d2_ref):
        deg = dacc_ref[0, :, :1] + dacc_ref[1, :, :1]
        dis = jnp.where(deg > 0.5, lax.rsqrt(deg), 0.0)
        acc = jnp.dot(x_ref[...], w_ref[...],
                      preferred_element_type=jnp.float32) + b_ref[...]
        out_ref[...] = acc
        outs_ref[...] = acc * dis
        dis_ref[...] = dis
        d2_ref[...] = dis * dis

    return pl.pallas_call(
        body,
        grid=(NPAD // bm,),
        in_specs=[
            pl.BlockSpec((bm, kd), lambda i: (i, 0)),
            pl.BlockSpec((kd, dout), lambda i: (0, 0)),
            pl.BlockSpec((1, dout), lambda i: (0, 0)),
            pl.BlockSpec((2, bm, 16), lambda i: (0, i, 0)),
        ],
        out_specs=[pl.BlockSpec((bm, dout), lambda i: (i, 0))] * 2
        + [pl.BlockSpec((bm, 1), lambda i: (i, 0))] * 2,
        out_shape=[jax.ShapeDtypeStruct((NPAD, dout), jnp.float32)] * 2
        + [jax.ShapeDtypeStruct((NPAD, 1), jnp.float32)] * 2,
    )(x, w, b.reshape(1, dout), deg_acc)


def _update(acc, a, b2d, dout):
    bm = 1024

    def body(acc_ref, a_ref, b_ref, o_ref):
        o_ref[...] = C1 * a_ref[...] + C2 * b_ref[...] * (acc_ref[0] + acc_ref[1])

    return pl.pallas_call(
        body,
        grid=(NPAD // bm,),
        in_specs=[
            pl.BlockSpec((2, bm, dout), lambda i: (0, i, 0)),
            pl.BlockSpec((bm, dout), lambda i: (i, 0)),
            pl.BlockSpec((bm, 1), lambda i: (i, 0)),
        ],
        out_specs=pl.BlockSpec((bm, dout), lambda i: (i, 0)),
        out_shape=jax.ShapeDtypeStruct((NPAD, dout), jnp.float32),
    )(acc, a, b2d)


def _block(prop, zs0, srcp, dstp, zer, ini, ini_s, dis2d, d22d, d):
    zs = zs0
    for _ in range(K_HOPS - 1):
        acc = prop(zs, srcp, dstp, zer)
        zs = _update(acc, ini_s, d22d, d)
    acc = prop(zs, srcp, dstp, zer)
    return _update(acc, ini, dis2d, d)


def kernel(features, edge_index, W1, b1, W2, b2):
    src = edge_index[0].astype(jnp.int32)
    dst = edge_index[1].astype(jnp.int32)
    srcp = jnp.concatenate(
        [src, jnp.full((EPAD - E,), N, jnp.int32)]).reshape(32, CPW, C)
    dstp = jnp.concatenate(
        [dst, jnp.full((EPAD - E,), NPAD - 1, jnp.int32)]).reshape(32, CPW, C)

    ones16 = jnp.broadcast_to(
        (jnp.arange(NPAD) < N).astype(jnp.float32)[:, None], (NPAD, 16))
    zer16 = jnp.zeros((RZT, 16), jnp.float32)
    zer64 = jnp.zeros((RZT, D_HID), jnp.float32)
    zer48 = jnp.zeros((RZT, 48), jnp.float32)

    deg_acc = _prop16(ones16, srcp, dstp, zer16)

    featp = jnp.pad(features, ((0, NPAD - N), (0, 0)))
    ini1, ini1_s, dis2d, d22d = _mlp(featp, W1, b1, deg_acc, D_HID)
    h = _block(_prop64, ini1_s, srcp, dstp, zer64, ini1, ini1_s, dis2d, d22d,
               D_HID)

    w2p = jnp.pad(W2, ((0, 0), (0, 1)))
    b2p = jnp.pad(b2, (0, 1))
    ini2, ini2_s, _, _ = _mlp(h, w2p, b2p, deg_acc, 48)
    out = _block(_prop48, ini2_s, srcp, dstp, zer48, ini2, ini2_s, dis2d, d22d,
                 48)
    return out[:N, :N_CLASSES]
